# Initial kernel scaffold; baseline (speedup 1.0000x reference)
import jax
import jax.numpy as jnp
from jax.experimental import pallas as pl


def kernel(outputs, y):
    raise NotImplementedError("write your pallas kernel here")


# speedup vs baseline: 2.2484x; 2.2484x over previous
import functools

import jax
import jax.numpy as jnp
import numpy as np
from jax import lax
from jax.experimental import pallas as pl
from jax.experimental.pallas import tpu as pltpu
from jax.experimental.pallas import tpu_sc as plsc

MAX_PAIRS = 500000
NUM_CORES = 2
NUM_SUBCORES = 16
NUM_WORKERS = NUM_CORES * NUM_SUBCORES
LANES = 16

PAIRS_PER_WORKER = -(-MAX_PAIRS // (NUM_WORKERS * LANES)) * LANES
PADDED_PAIRS = PAIRS_PER_WORKER * NUM_WORKERS
VECS_PER_WORKER = PAIRS_PER_WORKER // LANES

_IDX_CACHE = {}

_U32 = np.uint64(0xFFFFFFFF)


def _threefry2x32(k1, k2, x0, x1):
    k1 = np.uint64(k1)
    k2 = np.uint64(k2)
    x0 = np.asarray(x0, np.uint64)
    x1 = np.asarray(x1, np.uint64)
    ks = [k1 & _U32, k2 & _U32, (k1 ^ k2 ^ np.uint64(0x1BD11BDA)) & _U32]
    rot = ([13, 15, 26, 6], [17, 29, 16, 24])
    x0 = (x0 + ks[0]) & _U32
    x1 = (x1 + ks[1]) & _U32
    for i in range(5):
        for r in rot[i % 2]:
            x0 = (x0 + x1) & _U32
            x1 = ((x1 << np.uint64(r)) | (x1 >> np.uint64(32 - r))) & _U32
            x1 = x0 ^ x1
        x0 = (x0 + ks[(i + 1) % 3]) & _U32
        x1 = (x1 + ks[(i + 2) % 3] + np.uint64(i + 1)) & _U32
    return x0.astype(np.uint32), x1.astype(np.uint32)


def _randint_np(seed, shape, minval, maxval):
    k1 = np.uint32((seed >> 32) & 0xFFFFFFFF)
    k2 = np.uint32(seed & 0xFFFFFFFF)
    size = int(np.prod(shape))
    b1, b2 = _threefry2x32(k1, k2, np.zeros(2, np.uint64), np.arange(2, dtype=np.uint64))
    keys = np.stack([b1, b2], axis=1)
    cnt_hi = np.zeros(size, np.uint64)
    cnt_lo = np.arange(size, dtype=np.uint64)
    h1, h2 = _threefry2x32(keys[0][0], keys[0][1], cnt_hi, cnt_lo)
    higher = (h1 ^ h2).reshape(shape)
    h1, h2 = _threefry2x32(keys[1][0], keys[1][1], cnt_hi, cnt_lo)
    lower = (h1 ^ h2).reshape(shape)
    span = np.uint64(maxval - minval)
    mult = np.uint64(2**16) % span
    mult = (mult * mult) % span
    off = ((higher.astype(np.uint64) % span) * mult + lower.astype(np.uint64) % span) & _U32
    off = off % span
    return (np.int64(minval) + off.astype(np.int64)).astype(np.int32)


def _pair_indices(n):
    if n not in _IDX_CACHE:
        idx = _randint_np(42, (2, MAX_PAIRS), 0, n)
        pad = np.zeros((2, PADDED_PAIRS - MAX_PAIRS), np.int32)
        arr = np.concatenate([idx, pad], axis=1)
        _IDX_CACHE[n] = (arr[0], arr[1])
    return _IDX_CACHE[n]


def _make_sc_kernel():
    mesh = plsc.VectorSubcoreMesh(
        core_axis_name="c", subcore_axis_name="s", num_cores=NUM_CORES
    )
    B = PAIRS_PER_WORKER

    @functools.partial(
        pl.kernel,
        mesh=mesh,
        out_type=jax.ShapeDtypeStruct((2 * NUM_WORKERS, LANES), jnp.float32),
        scratch_types=[
            pltpu.VMEM((B,), jnp.int32),
            pltpu.VMEM((B,), jnp.int32),
            pltpu.VMEM((B,), jnp.float32),
            pltpu.VMEM((B,), jnp.float32),
            pltpu.VMEM((B,), jnp.float32),
            pltpu.VMEM((B,), jnp.float32),
            pltpu.VMEM((LANES,), jnp.float32),
            pltpu.VMEM((LANES,), jnp.float32),
            pltpu.SemaphoreType.DMA,
        ],
    )
    def ranking_loss_kernel(
        o_hbm, y_hbm, ii_hbm, jj_hbm, out_hbm,
        ii_v, jj_v, oi_v, oj_v, yi_v, yj_v, nv, dv, sem,
    ):
        wid = lax.axis_index("s") * NUM_CORES + lax.axis_index("c")
        base = wid * B
        pltpu.sync_copy(ii_hbm.at[pl.ds(base, B)], ii_v)
        pltpu.sync_copy(jj_hbm.at[pl.ds(base, B)], jj_v)
        c0 = pltpu.async_copy(o_hbm.at[ii_v], oi_v, sem)
        c1 = pltpu.async_copy(o_hbm.at[jj_v], oj_v, sem)
        c2 = pltpu.async_copy(y_hbm.at[ii_v], yi_v, sem)
        c3 = pltpu.async_copy(y_hbm.at[jj_v], yj_v, sem)
        c0.wait()
        c1.wait()
        c2.wait()
        c3.wait()

        def body(k, carry):
            num, den = carry
            sl = pl.ds(k * LANES, LANES)
            d_o = oi_v[sl] - oj_v[sl]
            d_y = yi_v[sl] - yj_v[sl]
            t = jnp.sign(d_y)
            num = num + jnp.maximum(0.0, -t * d_o)
            den = den + jnp.where(d_y != 0.0, 1.0, 0.0)
            return num, den

        zeros = jnp.zeros((LANES,), jnp.float32)
        num, den = lax.fori_loop(0, VECS_PER_WORKER, body, (zeros, zeros), unroll=4)
        nv[...] = num
        dv[...] = den
        pltpu.sync_copy(nv, out_hbm.at[wid])
        pltpu.sync_copy(dv, out_hbm.at[NUM_WORKERS + wid])

    return ranking_loss_kernel


_SC_KERNEL = _make_sc_kernel()


def kernel(outputs, y):
    o = outputs.reshape(-1)
    yy = y.reshape(-1)
    n = o.shape[0]
    ii_np, jj_np = _pair_indices(n)
    ii = jnp.asarray(ii_np)
    jj = jnp.asarray(jj_np)
    partials = _SC_KERNEL(o, yy, ii, jj)
    num = jnp.sum(partials[:NUM_WORKERS])
    den = jnp.sum(partials[NUM_WORKERS:])
    return num / den


# speedup vs baseline: 3.1983x; 1.4225x over previous
import functools

import jax
import jax.numpy as jnp
import numpy as np
from jax import lax
from jax.experimental import pallas as pl
from jax.experimental.pallas import tpu as pltpu
from jax.experimental.pallas import tpu_sc as plsc

MAX_PAIRS = 500000
NUM_CORES = 2
NUM_SUBCORES = 16
NUM_WORKERS = NUM_CORES * NUM_SUBCORES
LANES = 16

PAIRS_PER_WORKER = -(-MAX_PAIRS // (NUM_WORKERS * LANES)) * LANES
PADDED_PAIRS = PAIRS_PER_WORKER * NUM_WORKERS
VECS_PER_WORKER = PAIRS_PER_WORKER // LANES

_IDX_CACHE = {}

_U32 = np.uint64(0xFFFFFFFF)


def _threefry2x32(k1, k2, x0, x1):
    k1 = np.uint64(k1)
    k2 = np.uint64(k2)
    x0 = np.asarray(x0, np.uint64)
    x1 = np.asarray(x1, np.uint64)
    ks = [k1 & _U32, k2 & _U32, (k1 ^ k2 ^ np.uint64(0x1BD11BDA)) & _U32]
    rot = ([13, 15, 26, 6], [17, 29, 16, 24])
    x0 = (x0 + ks[0]) & _U32
    x1 = (x1 + ks[1]) & _U32
    for i in range(5):
        for r in rot[i % 2]:
            x0 = (x0 + x1) & _U32
            x1 = ((x1 << np.uint64(r)) | (x1 >> np.uint64(32 - r))) & _U32
            x1 = x0 ^ x1
        x0 = (x0 + ks[(i + 1) % 3]) & _U32
        x1 = (x1 + ks[(i + 2) % 3] + np.uint64(i + 1)) & _U32
    return x0.astype(np.uint32), x1.astype(np.uint32)


def _randint_np(seed, shape, minval, maxval):
    k1 = np.uint32((seed >> 32) & 0xFFFFFFFF)
    k2 = np.uint32(seed & 0xFFFFFFFF)
    size = int(np.prod(shape))
    b1, b2 = _threefry2x32(k1, k2, np.zeros(2, np.uint64), np.arange(2, dtype=np.uint64))
    keys = np.stack([b1, b2], axis=1)
    cnt_hi = np.zeros(size, np.uint64)
    cnt_lo = np.arange(size, dtype=np.uint64)
    h1, h2 = _threefry2x32(keys[0][0], keys[0][1], cnt_hi, cnt_lo)
    higher = (h1 ^ h2).reshape(shape)
    h1, h2 = _threefry2x32(keys[1][0], keys[1][1], cnt_hi, cnt_lo)
    lower = (h1 ^ h2).reshape(shape)
    span = np.uint64(maxval - minval)
    mult = np.uint64(2**16) % span
    mult = (mult * mult) % span
    off = ((higher.astype(np.uint64) % span) * mult + lower.astype(np.uint64) % span) & _U32
    off = off % span
    return (np.int64(minval) + off.astype(np.int64)).astype(np.int32)


def _pair_indices(n):
    if n not in _IDX_CACHE:
        idx = _randint_np(42, (2, MAX_PAIRS), 0, n)
        order = np.argsort(idx[0], kind="stable")
        ii_s = idx[0][order]
        jj_s = idx[1][order]
        rows = n // NUM_WORKERS
        counts = np.bincount(ii_s // rows, minlength=NUM_WORKERS)
        per_worker = int(-(-counts.max() // LANES) * LANES)
        li = np.zeros((NUM_WORKERS, per_worker), np.int32)
        jj = np.zeros((NUM_WORKERS, per_worker), np.int32)
        starts = np.concatenate([[0], np.cumsum(counts)])
        for w in range(NUM_WORKERS):
            c = counts[w]
            li[w, :c] = ii_s[starts[w]:starts[w + 1]] - w * rows
            jj[w, :c] = jj_s[starts[w]:starts[w + 1]]
            jj[w, c:] = w * rows
        _IDX_CACHE[n] = (li.reshape(-1), jj.reshape(-1), per_worker)
    return _IDX_CACHE[n]


def _make_sc_kernel(n, per_worker):
    mesh = plsc.VectorSubcoreMesh(
        core_axis_name="c", subcore_axis_name="s", num_cores=NUM_CORES
    )
    B = per_worker
    rows = n // NUM_WORKERS

    @functools.partial(
        pl.kernel,
        mesh=mesh,
        out_type=jax.ShapeDtypeStruct((2 * NUM_WORKERS, LANES), jnp.float32),
        compiler_params=pltpu.CompilerParams(needs_layout_passes=False),
        scratch_types=[
            pltpu.VMEM((rows,), jnp.float32),
            pltpu.VMEM((rows,), jnp.float32),
            pltpu.VMEM((B,), jnp.int32),
            pltpu.VMEM((B,), jnp.int32),
            pltpu.VMEM((B,), jnp.float32),
            pltpu.VMEM((B,), jnp.float32),
            pltpu.VMEM((LANES,), jnp.float32),
            pltpu.VMEM((LANES,), jnp.float32),
            pltpu.SemaphoreType.DMA,
            pltpu.SemaphoreType.DMA,
        ],
    )
    def ranking_loss_kernel(
        o_hbm, y_hbm, li_hbm, jj_hbm, out_hbm,
        or_v, yr_v, li_v, jj_v, oj_v, yj_v, nv, dv, sem, sem2,
    ):
        wid = lax.axis_index("s") * NUM_CORES + lax.axis_index("c")
        base = wid * B
        rbase = wid * rows
        pltpu.sync_copy(jj_hbm.at[pl.ds(base, B)], jj_v)
        c0 = pltpu.async_copy(o_hbm.at[jj_v], oj_v, sem)
        c1 = pltpu.async_copy(y_hbm.at[jj_v], yj_v, sem)
        c2 = pltpu.async_copy(o_hbm.at[pl.ds(rbase, rows)], or_v, sem2)
        c3 = pltpu.async_copy(y_hbm.at[pl.ds(rbase, rows)], yr_v, sem2)
        c4 = pltpu.async_copy(li_hbm.at[pl.ds(base, B)], li_v, sem2)
        c0.wait()
        c1.wait()
        c2.wait()
        c3.wait()
        c4.wait()

        def body(k, carry):
            num, den = carry
            sl = pl.ds(k * LANES, LANES)
            liv = li_v[sl]
            o_i = plsc.load_gather(or_v, [liv])
            y_i = plsc.load_gather(yr_v, [liv])
            d_o = o_i - oj_v[sl]
            d_y = y_i - yj_v[sl]
            t = jnp.sign(d_y)
            num = num + jnp.maximum(0.0, -t * d_o)
            den = den + jnp.where(d_y != 0.0, 1.0, 0.0)
            return num, den

        zeros = jnp.zeros((LANES,), jnp.float32)
        num, den = lax.fori_loop(0, B // LANES, body, (zeros, zeros), unroll=4)
        nv[...] = num
        dv[...] = den
        pltpu.sync_copy(nv, out_hbm.at[wid])
        pltpu.sync_copy(dv, out_hbm.at[NUM_WORKERS + wid])

    return ranking_loss_kernel


_KERNEL_CACHE = {}


def kernel(outputs, y):
    o = outputs.reshape(-1)
    yy = y.reshape(-1)
    n = o.shape[0]
    li_np, jj_np, per_worker = _pair_indices(n)
    if n not in _KERNEL_CACHE:
        _KERNEL_CACHE[n] = _make_sc_kernel(n, per_worker)
    li = jnp.asarray(li_np)
    jj = jnp.asarray(jj_np)
    partials = _KERNEL_CACHE[n](o, yy, li, jj)
    num = jnp.sum(partials[:NUM_WORKERS])
    den = jnp.sum(partials[NUM_WORKERS:])
    return num / den
